# baseline (speedup 1.0000x reference)
import functools

import jax
import jax.numpy as jnp
from jax import lax
from jax.experimental import pallas as pl
from jax.experimental.pallas import tpu as pltpu
from jax.experimental.pallas import tpu_sc as plsc

NUM_EMB = 1000000
DIM = 32
TOTAL = 16384 * 50

_NC = 2
_NS = 16
_NW = _NC * _NS
_BPW = TOTAL // _NW
_CHUNK = 1280
_NCHUNK = _BPW // _CHUNK


def _emb_kernel(idx_hbm, table_hbm, out_hbm, idx_v, rows_a, rows_b,
                sem_idx, sem_g, sem_w):
    wid = lax.axis_index("s") * _NC + lax.axis_index("c")
    base = wid * _BPW
    pltpu.async_copy(idx_hbm.at[pl.ds(base, _BPW)], idx_v, sem_idx).wait()

    bufs = (rows_a, rows_b)
    writes = [None] * _NCHUNK
    for t in range(_NCHUNK):
        buf = bufs[t % 2]
        if t >= 2:
            writes[t - 2].wait()
        pltpu.async_copy(
            table_hbm.at[idx_v.at[pl.ds(t * _CHUNK, _CHUNK)]], buf, sem_g
        ).wait()
        writes[t] = pltpu.async_copy(
            buf, out_hbm.at[pl.ds(base + t * _CHUNK, _CHUNK)], sem_w)
    writes[_NCHUNK - 2].wait()
    writes[_NCHUNK - 1].wait()


@jax.jit
def _embedding_lookup(idx_flat, weight):
    mesh = plsc.VectorSubcoreMesh(core_axis_name="c", subcore_axis_name="s")
    f = functools.partial(
        pl.kernel,
        mesh=mesh,
        out_type=jax.ShapeDtypeStruct((TOTAL, DIM), jnp.float32),
        scratch_types=[
            pltpu.VMEM((_BPW,), jnp.int32),
            pltpu.VMEM((_CHUNK, DIM), jnp.float32),
            pltpu.VMEM((_CHUNK, DIM), jnp.float32),
            pltpu.SemaphoreType.DMA,
            pltpu.SemaphoreType.DMA,
            pltpu.SemaphoreType.DMA,
        ],
        compiler_params=pltpu.CompilerParams(use_tc_tiling_on_sc=False),
    )(_emb_kernel)
    return f(idx_flat, weight)


def kernel(x, weight):
    idx_flat = jnp.minimum(x.astype(jnp.int32), NUM_EMB - 1).reshape(-1)
    w = jnp.minimum(weight, jnp.float32(3.4e38))
    out = _embedding_lookup(idx_flat, w)
    return out.reshape(x.shape[0], x.shape[1], DIM)


# speedup vs baseline: 1.1907x; 1.1907x over previous
import functools

import jax
import jax.numpy as jnp
from jax import lax
from jax.experimental import pallas as pl
from jax.experimental.pallas import tpu as pltpu
from jax.experimental.pallas import tpu_sc as plsc

NUM_EMB = 1000000
DIM = 32
TOTAL = 16384 * 50

_NC = 2
_NS = 16
_NW = _NC * _NS
_BPW = TOTAL // _NW
_CHUNK = 1280
_NCHUNK = _BPW // _CHUNK


def _emb_kernel(idx_hbm, table_hbm, out_hbm, idx_v, rows_a, rows_b,
                sem_idx, sem_g, sem_w):
    wid = lax.axis_index("s") * _NC + lax.axis_index("c")
    base = wid * _BPW
    pltpu.async_copy(idx_hbm.at[pl.ds(base, _BPW)], idx_v, sem_idx).wait()

    bufs = (rows_a, rows_b)
    writes = [None] * _NCHUNK
    for t in range(_NCHUNK):
        buf = bufs[t % 2]
        if t >= 2:
            writes[t - 2].wait()
        pltpu.async_copy(
            table_hbm.at[idx_v.at[pl.ds(t * _CHUNK, _CHUNK)]], buf, sem_g
        ).wait()
        writes[t] = pltpu.async_copy(
            buf, out_hbm.at[pl.ds(base + t * _CHUNK, _CHUNK)], sem_w)
    writes[_NCHUNK - 2].wait()
    writes[_NCHUNK - 1].wait()


@jax.jit
def _embedding_lookup(idx_flat, weight):
    mesh = plsc.VectorSubcoreMesh(core_axis_name="c", subcore_axis_name="s")
    f = functools.partial(
        pl.kernel,
        mesh=mesh,
        out_type=jax.ShapeDtypeStruct((TOTAL, DIM), jnp.float32),
        scratch_types=[
            pltpu.VMEM((_BPW,), jnp.int32),
            pltpu.VMEM((_CHUNK, DIM), jnp.float32),
            pltpu.VMEM((_CHUNK, DIM), jnp.float32),
            pltpu.SemaphoreType.DMA,
            pltpu.SemaphoreType.DMA,
            pltpu.SemaphoreType.DMA,
        ],
        compiler_params=pltpu.CompilerParams(use_tc_tiling_on_sc=False),
    )(_emb_kernel)
    return f(idx_flat, weight)


def kernel(x, weight):
    idx_flat = jnp.minimum(x.astype(jnp.int32), NUM_EMB - 1).reshape(-1)
    out = _embedding_lookup(idx_flat, weight)
    return out.reshape(x.shape[0], x.shape[1], DIM)


# speedup vs baseline: 1.4746x; 1.2384x over previous
import functools

import jax
import jax.numpy as jnp
from jax import lax
from jax.experimental import pallas as pl
from jax.experimental.pallas import tpu as pltpu
from jax.experimental.pallas import tpu_sc as plsc

NUM_EMB = 1000000
DIM = 32
B = 16384
J = 50

_NS = 16
_SPT = B // _NS
_JPC = J // 2


def _emb_kernel(idx_hbm, table_hbm, out_hbm, xbuf, rows_v, tbuf, sem_i, sem_g):
    cid = lax.axis_index("c")
    sid = lax.axis_index("s")
    pltpu.async_copy(
        idx_hbm.at[pl.ds(cid * _JPC, _JPC), pl.ds(sid * _SPT, _SPT)],
        xbuf, sem_i).wait()

    def chunk_body(ci, carry):
        j = cid * _JPC + ci
        pltpu.async_copy(
            table_hbm.at[xbuf.at[ci]], rows_v, sem_g).wait()

        def c_body(c, carry2):
            cvec = jnp.full((16,), c, dtype=jnp.int32)

            def i_body(i, carry3):
                r = i * 16 + lax.iota(jnp.int32, 16)
                tbuf[c, pl.ds(i * 16, 16)] = plsc.load_gather(
                    rows_v, [r, cvec])
                return carry3

            lax.fori_loop(0, _SPT // 16, i_body, 0)
            return carry2

        lax.fori_loop(0, DIM, c_body, 0)
        pltpu.sync_copy(tbuf, out_hbm.at[j, :, pl.ds(sid * _SPT, _SPT)])
        return carry

    lax.fori_loop(0, _JPC, chunk_body, 0)


@jax.jit
def _embedding_lookup(idx2d, weight):
    mesh = plsc.VectorSubcoreMesh(core_axis_name="c", subcore_axis_name="s")
    f = functools.partial(
        pl.kernel,
        mesh=mesh,
        out_type=jax.ShapeDtypeStruct((J, DIM, B), jnp.float32),
        scratch_types=[
            pltpu.VMEM((_JPC, _SPT), jnp.int32),
            pltpu.VMEM((_SPT, DIM), jnp.float32),
            pltpu.VMEM((DIM, _SPT), jnp.float32),
            pltpu.SemaphoreType.DMA,
            pltpu.SemaphoreType.DMA,
        ],
        compiler_params=pltpu.CompilerParams(
            use_tc_tiling_on_sc=False, needs_layout_passes=False),
    )(_emb_kernel)
    return f(idx2d, weight)


def kernel(x, weight):
    idx2d = jnp.minimum(x.astype(jnp.int32), NUM_EMB - 1).T
    out = _embedding_lookup(idx2d, weight)
    return out.transpose(2, 0, 1)


# speedup vs baseline: 1.5499x; 1.0511x over previous
import functools

import jax
import jax.numpy as jnp
from jax import lax
from jax.experimental import pallas as pl
from jax.experimental.pallas import tpu as pltpu
from jax.experimental.pallas import tpu_sc as plsc

NUM_EMB = 1000000
DIM = 32
B = 16384
J = 50

_NS = 16
_SPT = B // _NS
_JPC = J // 2


def _emb_kernel(idx_hbm, table_hbm, out_hbm, xbuf, rows_a, rows_b, tbuf,
                sem_i, sem_a, sem_b):
    cid = lax.axis_index("c")
    sid = lax.axis_index("s")
    pltpu.async_copy(
        idx_hbm.at[pl.ds(cid * _JPC, _JPC), pl.ds(sid * _SPT, _SPT)],
        xbuf, sem_i).wait()

    bufs = (rows_a, rows_b)
    sems = (sem_a, sem_b)
    gathers = [None] * _JPC
    gathers[0] = pltpu.async_copy(
        table_hbm.at[xbuf.at[0]], bufs[0], sems[0])
    for ci in range(_JPC):
        rows_v = bufs[ci % 2]
        if ci + 1 < _JPC:
            gathers[ci + 1] = pltpu.async_copy(
                table_hbm.at[xbuf.at[ci + 1]], bufs[(ci + 1) % 2],
                sems[(ci + 1) % 2])
        gathers[ci].wait()

        def c_body(c, carry2, rows_v=rows_v):
            cvec = jnp.full((16,), c, dtype=jnp.int32)

            def i_body(i, carry3):
                for u in range(4):
                    r = (i * 4 + u) * 16 + lax.iota(jnp.int32, 16)
                    tbuf[c, pl.ds((i * 4 + u) * 16, 16)] = plsc.load_gather(
                        rows_v, [r, cvec])
                return carry3

            lax.fori_loop(0, _SPT // 64, i_body, 0)
            return carry2

        lax.fori_loop(0, DIM, c_body, 0)
        pltpu.sync_copy(
            tbuf, out_hbm.at[cid * _JPC + ci, :, pl.ds(sid * _SPT, _SPT)])


@jax.jit
def _embedding_lookup(idx2d, weight):
    mesh = plsc.VectorSubcoreMesh(core_axis_name="c", subcore_axis_name="s")
    f = functools.partial(
        pl.kernel,
        mesh=mesh,
        out_type=jax.ShapeDtypeStruct((J, DIM, B), jnp.float32),
        scratch_types=[
            pltpu.VMEM((_JPC, _SPT), jnp.int32),
            pltpu.VMEM((_SPT, DIM), jnp.float32),
            pltpu.VMEM((_SPT, DIM), jnp.float32),
            pltpu.VMEM((DIM, _SPT), jnp.float32),
            pltpu.SemaphoreType.DMA,
            pltpu.SemaphoreType.DMA,
            pltpu.SemaphoreType.DMA,
        ],
        compiler_params=pltpu.CompilerParams(
            use_tc_tiling_on_sc=False, needs_layout_passes=False),
    )(_emb_kernel)
    return f(idx2d, weight)


def kernel(x, weight):
    idx2d = jnp.minimum(x.astype(jnp.int32), NUM_EMB - 1).T
    out = _embedding_lookup(idx2d, weight)
    return out.transpose(2, 0, 1)
